# baseline (speedup 1.0000x reference)
import functools

import jax
import jax.numpy as jnp
from jax import lax
from jax.experimental import pallas as pl
from jax.experimental.pallas import tpu as pltpu
from jax.experimental.pallas import tpu_sc as plsc

CTX = 1000
B, S = 1024, 50
ROWS = B * S
NW = 32
RPW = ROWS // NW
ZCH = 32
NCH = RPW // ZCH
ZW = ZCH * CTX
L = 16
WIN = 12


def _body(seq_hbm, out_hbm, idx_v, idx2d, stage, ones_v, zeros_sh,
          semz, sems):
    cid = lax.axis_index("c")
    sid = lax.axis_index("s")
    wid = sid * 2 + cid
    row0 = wid * RPW

    pltpu.sync_copy(seq_hbm.at[pl.ds(row0, RPW)], idx_v)

    zero16 = jnp.zeros((L,), jnp.float32)
    one16 = jnp.full((L,), 1.0, jnp.float32)

    ZUNROLL = 16
    def _zero_body(i, carry):
        base = i * (ZUNROLL * L)
        for k in range(ZUNROLL):
            stage[pl.ds(base + k * L, L)] = zero16
        return carry
    lax.fori_loop(0, ZW // (ZUNROLL * L), _zero_body, 0)
    pltpu.sync_copy(stage, zeros_sh.at[sid])
    for o in range(0, ZCH, L):
        ones_v[pl.ds(o, L)] = one16

    iota_ctx = lax.iota(jnp.int32, L) * CTX
    row0k = row0 * CTX
    for c in range(NCH):
        for o in range(0, ZCH, L):
            g16 = idx_v[pl.ds(c * ZCH + o, L)]
            idx2d[c, pl.ds(o, L)] = g16 + iota_ctx + (row0k + (c * ZCH + o) * CTX)

    my_zeros = zeros_sh.at[sid]
    hz = [None] * NCH
    hs = [None] * NCH
    for c in range(NCH):
        dst = out_hbm.at[pl.ds(row0k + c * ZW, ZW)]
        hz[c] = pltpu.async_copy(my_zeros, dst, semz)
        if c >= WIN:
            p = c - WIN
            hz[p].wait()
            hs[p] = pltpu.async_copy(ones_v, out_hbm.at[idx2d.at[p]], sems)
    for p in range(NCH - WIN, NCH):
        hz[p].wait()
        hs[p] = pltpu.async_copy(ones_v, out_hbm.at[idx2d.at[p]], sems)
    for p in range(NCH):
        hs[p].wait()


@jax.jit
def _onehot_sc(seq_flat):
    kern = functools.partial(
        pl.kernel,
        mesh=plsc.VectorSubcoreMesh(core_axis_name="c", subcore_axis_name="s"),
        out_type=jax.ShapeDtypeStruct((ROWS * CTX,), jnp.float32),
        scratch_types=[
            pltpu.VMEM((RPW,), jnp.int32),
            pltpu.VMEM((NCH, ZCH), jnp.int32),
            pltpu.VMEM((ZW,), jnp.float32),
            pltpu.VMEM((ZCH,), jnp.float32),
            pltpu.VMEM_SHARED((16, ZW), jnp.float32),
            pltpu.SemaphoreType.DMA,
            pltpu.SemaphoreType.DMA,
        ],
        compiler_params=pltpu.CompilerParams(needs_layout_passes=False),
    )(_body)
    return kern(seq_flat)


def kernel(sequence):
    seq_flat = sequence.reshape(ROWS).astype(jnp.int32)
    out = _onehot_sc(seq_flat)
    return out.reshape(B, S, CTX)


# speedup vs baseline: 1.0710x; 1.0710x over previous
import functools

import jax
import jax.numpy as jnp
from jax import lax
from jax.experimental import pallas as pl
from jax.experimental.pallas import tpu as pltpu
from jax.experimental.pallas import tpu_sc as plsc

CTX = 1000
B, S = 1024, 50
ROWS = B * S
NW = 32
RPW = ROWS // NW
CHUNK = 32
NCHUNK = RPW // CHUNK
PAIRW = 2 * CTX
NPAIR = ROWS // 2
CP = CHUNK // 2
L = 16


def _body(seq_hbm, out_hbm, idx_v, buf0, buf1, sem0, sem1):
    cid = lax.axis_index("c")
    sid = lax.axis_index("s")
    wid = sid * 2 + cid
    row0 = wid * RPW

    pltpu.sync_copy(seq_hbm.at[pl.ds(row0, RPW)], idx_v)

    zero16 = jnp.zeros((L,), jnp.float32)
    one16 = jnp.full((L,), 1.0, jnp.float32)
    iota16 = lax.iota(jnp.int32, L)

    def _zero_body(i, carry):
        base = i * L
        for p in range(CP):
            buf0[p, pl.ds(base, L)] = zero16
            buf1[p, pl.ds(base, L)] = zero16
        return carry
    lax.fori_loop(0, PAIRW // L, _zero_body, 0)

    bufs = (buf0, buf1)
    sems = (sem0, sem1)

    def _buf_idx(c, o):
        idxs = idx_v[pl.ds(c * CHUNK + o, L)]
        r = iota16 + o
        rows = r >> 1
        cols = (r & 1) * CTX + idxs
        return rows, cols

    pair0 = wid * (RPW // 2)
    handles = [None, None]
    pending = [None, None]
    for c in range(NCHUNK):
        bsel = c & 1
        buf = bufs[bsel]
        if handles[bsel] is not None:
            handles[bsel].wait()
            pc = pending[bsel]
            for o in range(0, CHUNK, L):
                rows, cols = _buf_idx(pc, o)
                plsc.store_scatter(buf, [rows, cols], zero16)
        for o in range(0, CHUNK, L):
            rows, cols = _buf_idx(c, o)
            plsc.store_scatter(buf, [rows, cols], one16)
        pairs = iota16 + (pair0 + c * CP)
        handles[bsel] = pltpu.async_copy(buf, out_hbm.at[pairs], sems[bsel])
        pending[bsel] = c
    handles[0].wait()
    handles[1].wait()


@jax.jit
def _onehot_sc(seq_flat):
    kern = functools.partial(
        pl.kernel,
        mesh=plsc.VectorSubcoreMesh(core_axis_name="c", subcore_axis_name="s"),
        out_type=jax.ShapeDtypeStruct((NPAIR, PAIRW), jnp.float32),
        scratch_types=[
            pltpu.VMEM((RPW,), jnp.int32),
            pltpu.VMEM((CP, PAIRW), jnp.float32),
            pltpu.VMEM((CP, PAIRW), jnp.float32),
            pltpu.SemaphoreType.DMA,
            pltpu.SemaphoreType.DMA,
        ],
        compiler_params=pltpu.CompilerParams(
            needs_layout_passes=False, use_tc_tiling_on_sc=False),
    )(_body)
    return kern(seq_flat)


def kernel(sequence):
    seq_flat = sequence.reshape(ROWS).astype(jnp.int32)
    out = _onehot_sc(seq_flat)
    return out.reshape(B, S, CTX)
